# baseline (speedup 1.0000x reference)
import functools

import jax
import jax.numpy as jnp
from jax import lax
from jax.experimental import pallas as pl
from jax.experimental.pallas import tpu as pltpu
from jax.experimental.pallas import tpu_sc as plsc

NC = 2
NS = 16
NW = NC * NS

G = 128
CH = 512


def _gather_body(ids_hbm, table_hbm, out_hbm, idx_v, rows_v, sem):
    wid = lax.axis_index("s") * NC + lax.axis_index("c")
    rw = idx_v.shape[0] * idx_v.shape[1]
    base = wid * rw
    pltpu.sync_copy(ids_hbm.at[wid], idx_v)
    k = CH // G

    def chunk(c, _):
        descs = [
            pltpu.async_copy(
                table_hbm.at[idx_v.at[c * k + j]],
                rows_v.at[pl.ds(j * G, G)],
                sem,
            )
            for j in range(k)
        ]
        for d in descs:
            d.wait()
        pltpu.sync_copy(rows_v, out_hbm.at[pl.ds(base + c * CH, CH)])
        return 0

    lax.fori_loop(0, rw // CH, chunk, 0)


def _sc_gather(ids, emb_table):
    n = ids.shape[0]
    hid = emb_table.shape[1]
    rw = n // NW
    ids3 = ids.reshape(NW, rw // G, G)
    kern = functools.partial(
        pl.kernel,
        out_type=jax.ShapeDtypeStruct((n, hid), jnp.float32),
        mesh=plsc.VectorSubcoreMesh(core_axis_name="c", subcore_axis_name="s"),
        scratch_types=[
            pltpu.VMEM((rw // G, G), jnp.int32),
            pltpu.VMEM((CH, hid), jnp.float32),
            pltpu.SemaphoreType.DMA,
        ],
        compiler_params=pltpu.CompilerParams(use_tc_tiling_on_sc=False),
    )(_gather_body)
    return kern(ids3, emb_table)


def _proj_body(x_ref, m_ref, w_ref, b_ref, o_ref):
    x = x_ref[...]
    y = lax.dot_general(
        x, w_ref[...], (((1,), (1,)), ((), ())),
        preferred_element_type=jnp.float32,
    )
    y = y + b_ref[...]
    o_ref[...] = y * m_ref[...].astype(jnp.float32)


def _tc_project(rows, mask, W, b, blk=4096):
    n, hid = rows.shape
    grid = n // blk
    return pl.pallas_call(
        _proj_body,
        grid=(grid,),
        in_specs=[
            pl.BlockSpec((blk, hid), lambda i: (i, 0)),
            pl.BlockSpec((blk, 1), lambda i: (i, 0)),
            pl.BlockSpec((hid, hid), lambda i: (0, 0)),
            pl.BlockSpec((1, hid), lambda i: (0, 0)),
        ],
        out_specs=pl.BlockSpec((blk, hid), lambda i: (i, 0)),
        out_shape=jax.ShapeDtypeStruct((n, hid), jnp.float32),
    )(rows, mask, W, b)


def kernel(token_ids, attention_mask, emb_table, W, b):
    bsz, seq = token_ids.shape
    hid = emb_table.shape[1]
    n = bsz * seq
    ids = (token_ids * attention_mask).reshape(n)
    rows = _sc_gather(ids, emb_table)
    mask2 = attention_mask.reshape(n, 1)
    out = _tc_project(rows, mask2, W, b.reshape(1, hid))
    return out.reshape(bsz, seq, hid)
